# Initial kernel scaffold; baseline (speedup 1.0000x reference)
import jax
import jax.numpy as jnp
from jax.experimental import pallas as pl


def kernel(feat_erb, state):
    raise NotImplementedError("write your pallas kernel here")


# speedup vs baseline: 41.4465x; 41.4465x over previous
import math

import jax
import jax.numpy as jnp
import numpy as np
from jax.experimental import pallas as pl
from jax.experimental.pallas import tpu as pltpu


def _norm_alpha(sample_rate=8000, hop_size=80, norm_tau=1.0):
    a_ = math.exp(-(hop_size / sample_rate) / norm_tau)
    precision = 3
    a = 1.0
    while a >= 1.0:
        a = round(a_, precision)
        precision += 1
    return a


_ALPHA = _norm_alpha()

_L = 128
_T_BLK = 2048
_B_BLK = 4


def _decay_matrix(alpha, n):
    i = np.arange(n)[:, None]
    j = np.arange(n)[None, :]
    m = np.where(j <= i, (1.0 - alpha) * np.power(alpha, i - j), 0.0)
    return m.astype(np.float32)


def _ema_body(x_ref, m_ref, s0_ref, o_ref, fs_ref):
    tstep = pl.program_id(1)

    @pl.when(tstep == 0)
    def _init():
        fs_ref[...] = jnp.broadcast_to(s0_ref[...], fs_ref.shape)

    m = m_ref[...]
    row = jax.lax.broadcasted_iota(jnp.int32, (_L, 1), 0).astype(jnp.float32)
    pvec = jnp.exp(np.float32(math.log(_ALPHA)) * (row + 1.0))

    n_chunks = _T_BLK // _L
    for bb in range(_B_BLK):
        h = fs_ref[bb]
        for c in range(n_chunks):
            xc = x_ref[bb, pl.ds(c * _L, _L), :]
            s = jnp.dot(m, xc, preferred_element_type=jnp.float32) + pvec * h
            o_ref[bb, pl.ds(c * _L, _L), :] = (xc - s) * np.float32(1.0 / 40.0)
            h = s[_L - 1:_L, :]
        fs_ref[bb] = h


def kernel(feat_erb, state):
    b, c, t, f = feat_erb.shape
    bc = b * c
    x = feat_erb.reshape(bc, t, f)
    s0 = state.astype(feat_erb.dtype).reshape(1, 1, f)
    m = jnp.asarray(_decay_matrix(_ALPHA, _L))

    grid = (bc // _B_BLK, t // _T_BLK)
    out, fstate = pl.pallas_call(
        _ema_body,
        grid=grid,
        in_specs=[
            pl.BlockSpec((_B_BLK, _T_BLK, f), lambda i, j: (i, j, 0)),
            pl.BlockSpec((_L, _L), lambda i, j: (0, 0)),
            pl.BlockSpec((1, 1, f), lambda i, j: (0, 0, 0)),
        ],
        out_specs=[
            pl.BlockSpec((_B_BLK, _T_BLK, f), lambda i, j: (i, j, 0)),
            pl.BlockSpec((_B_BLK, 1, f), lambda i, j: (i, 0, 0)),
        ],
        out_shape=[
            jax.ShapeDtypeStruct((bc, t, f), feat_erb.dtype),
            jax.ShapeDtypeStruct((bc, 1, f), feat_erb.dtype),
        ],
        compiler_params=pltpu.CompilerParams(
            dimension_semantics=("parallel", "arbitrary"),
        ),
        name="erb_ema",
    )(x, m, s0)

    return out.reshape(b, c, t, f), fstate.reshape(b, c, f)
